# Initial kernel scaffold; baseline (speedup 1.0000x reference)
import jax
import jax.numpy as jnp
from jax.experimental import pallas as pl


def kernel(x, a, e, fgn_w0, fgn_b0, root0, bias0, fgn_w, fgn_b, root, bias, dense_w, dense_b):
    raise NotImplementedError("write your pallas kernel here")


# speedup vs baseline: 58.7031x; 58.7031x over previous
import jax
import jax.numpy as jnp
from jax.experimental import pallas as pl

B, N, F0, S, U, L, NOUT = 8, 32, 32, 16, 64, 4, 19


def _net_kernel(x_ref, a_ref, e_ref,
                wt0_ref, bt0_ref, root0_ref, bias0_ref,
                wt_ref, bt_ref, root_ref, bias_ref,
                dw_ref, db_ref, out_ref):
    h = x_ref[0, :, :F0]
    mask = x_ref[0, :, F0:]
    a = a_ref[0]

    a_tiled = jnp.concatenate([a] * S, axis=1)
    ae2 = a_tiled * e_ref[0]

    def ecc(h, wt_s, bt, rk, bk):
        wh2 = jnp.concatenate(
            [jnp.dot(h, w, preferred_element_type=jnp.float32) for w in wt_s],
            axis=0)
        out = jnp.dot(ae2, wh2, preferred_element_type=jnp.float32)
        bh = jnp.dot(h, bt, preferred_element_type=jnp.float32)
        out += jnp.dot(a, bh, preferred_element_type=jnp.float32)
        out += jnp.dot(h, rk, preferred_element_type=jnp.float32) + bk
        return jnp.maximum(out, 0.0)

    h = ecc(h, [wt0_ref[s] for s in range(S)],
            bt0_ref[...], root0_ref[...], bias0_ref[...])
    for l in range(L - 1):
        h = ecc(h, [wt_ref[l, s] for s in range(S)],
                bt_ref[l], root_ref[l], bias_ref[l][None, :])

    pooled = jnp.sum(h * mask, axis=0, keepdims=True)
    out_ref[0] = jnp.dot(pooled, dw_ref[...],
                         preferred_element_type=jnp.float32) + db_ref[...]


def kernel(x, a, e, fgn_w0, fgn_b0, root0, bias0, fgn_w, fgn_b, root, bias, dense_w, dense_b):
    e_l = e.transpose(0, 1, 3, 2).reshape(B, N, S * N)
    wt0 = fgn_w0.reshape(S, U, F0).transpose(0, 2, 1)
    bt0 = fgn_b0.reshape(U, F0).T
    wt = fgn_w.reshape(L - 1, S, U, U).transpose(0, 1, 3, 2)
    bt = fgn_b.reshape(L - 1, U, U).transpose(0, 2, 1)
    bias0_2d = bias0[None, :]
    db = dense_b[None, :]

    rep = lambda shape: pl.BlockSpec(shape, lambda b: (0,) * len(shape))
    grid_spec = pl.GridSpec(
        grid=(B,),
        in_specs=[
            pl.BlockSpec((1, N, F0 + 1), lambda b: (b, 0, 0)),
            pl.BlockSpec((1, N, N), lambda b: (b, 0, 0)),
            pl.BlockSpec((1, N, S * N), lambda b: (b, 0, 0)),
            rep((S, F0, U)), rep((F0, U)), rep((F0, U)), rep((1, U)),
            rep((L - 1, S, U, U)), rep((L - 1, U, U)), rep((L - 1, U, U)),
            rep((L - 1, U)),
            rep((U, NOUT)), rep((1, NOUT)),
        ],
        out_specs=pl.BlockSpec((1, 1, NOUT), lambda b: (b, 0, 0)),
    )
    out = pl.pallas_call(
        _net_kernel,
        grid_spec=grid_spec,
        out_shape=jax.ShapeDtypeStruct((B, 1, NOUT), jnp.float32),
    )(x, a, e_l, wt0, bt0, root0, bias0_2d, wt, bt, root, bias, dense_w, db)
    return out.reshape(B, NOUT)


# speedup vs baseline: 93.9584x; 1.6006x over previous
import jax
import jax.numpy as jnp
from jax.experimental import pallas as pl

B, N, F0, S, U, L, NOUT = 8, 32, 32, 16, 64, 4, 19


def _net_kernel(x_ref, a_ref, e_ref,
                wt0_ref, bt0_ref, root0_ref, bias0_ref,
                wt_ref, bt_ref, root_ref, bias_ref,
                dw_ref, db_ref, out_ref):
    f32 = jnp.float32
    h_all = jnp.concatenate([x_ref[b, :, :F0] for b in range(B)], axis=0)
    mask_all = jnp.concatenate([x_ref[b, :, F0:] for b in range(B)], axis=0)

    ae2 = []
    for b in range(B):
        a_b = a_ref[b]
        ae2.append(jnp.concatenate([a_b] * S, axis=1) * e_ref[b])

    def ecc(h_all, wt_s, bt, rk, bk):
        wh = [jnp.dot(h_all, w, preferred_element_type=f32) for w in wt_s]
        bh = jnp.dot(h_all, bt, preferred_element_type=f32)
        rooted = jnp.dot(h_all, rk, preferred_element_type=f32) + bk
        outs = []
        for b in range(B):
            lo = b * N
            wh2 = jnp.concatenate([w[lo:lo + N] for w in wh], axis=0)
            agg = jnp.dot(ae2[b], wh2, preferred_element_type=f32)
            agg += jnp.dot(a_ref[b], bh[lo:lo + N], preferred_element_type=f32)
            outs.append(jnp.maximum(agg + rooted[lo:lo + N], 0.0))
        return jnp.concatenate(outs, axis=0)

    h_all = ecc(h_all, [wt0_ref[s] for s in range(S)],
                bt0_ref[...], root0_ref[...], bias0_ref[...])
    for l in range(L - 1):
        h_all = ecc(h_all, [wt_ref[l, s] for s in range(S)],
                    bt_ref[l], root_ref[l], bias_ref[l][None, :])

    hm = h_all * mask_all
    pooled = jnp.concatenate(
        [jnp.sum(hm[b * N:(b + 1) * N], axis=0, keepdims=True)
         for b in range(B)], axis=0)
    out_ref[...] = jnp.dot(pooled, dw_ref[...],
                           preferred_element_type=f32) + db_ref[...]


def kernel(x, a, e, fgn_w0, fgn_b0, root0, bias0, fgn_w, fgn_b, root, bias, dense_w, dense_b):
    e_l = e.transpose(0, 1, 3, 2).reshape(B, N, S * N)
    wt0 = fgn_w0.reshape(S, U, F0).transpose(0, 2, 1)
    bt0 = fgn_b0.reshape(U, F0).T
    wt = fgn_w.reshape(L - 1, S, U, U).transpose(0, 1, 3, 2)
    bt = fgn_b.reshape(L - 1, U, U).transpose(0, 2, 1)
    bias0_2d = bias0[None, :]
    db = dense_b[None, :]

    return pl.pallas_call(
        _net_kernel,
        out_shape=jax.ShapeDtypeStruct((B, NOUT), jnp.float32),
    )(x, a, e_l, wt0, bt0, root0, bias0_2d, wt, bt, root, bias, dense_w, db)
